# baseline (speedup 1.0000x reference)
import jax
import jax.numpy as jnp
from jax import lax
from jax.experimental import pallas as pl
from jax.experimental.pallas import tpu as pltpu
from jax.experimental.pallas import tpu_sc as plsc

D = 128
L = 8
B = 128
V = 100000
MAX_NORM = 1.0

NC, NS = 2, 16
NW = NC * NS

TV = 2000
NSPLIT = 2
TH = TV // NSPLIT


def _gather_body(table_hbm, idx_hbm, out_hbm, idx_v, rows_v, sem):
    wid = lax.axis_index("s") * NC + lax.axis_index("c")
    n = idx_v.shape[0]
    base = wid * n
    pltpu.sync_copy(idx_hbm.at[pl.ds(base, n)], idx_v)
    pltpu.async_copy(table_hbm.at[idx_v], rows_v, sem).wait()
    pltpu.sync_copy(rows_v, out_hbm.at[pl.ds(base, n)])


def _sc_gather(table, idx_flat):
    n_tok = idx_flat.shape[0]
    per_w = n_tok // NW
    mesh = plsc.VectorSubcoreMesh(
        core_axis_name="c", subcore_axis_name="s", num_cores=NC, num_subcores=NS
    )
    return pl.kernel(
        _gather_body,
        out_type=jax.ShapeDtypeStruct((n_tok, D), jnp.float32),
        mesh=mesh,
        scratch_types=[
            pltpu.VMEM((per_w,), jnp.int32),
            pltpu.VMEM((per_w, D), jnp.float32),
            pltpu.SemaphoreType.DMA,
        ],
    )(table, idx_flat)


def _out_copy(scr_ref, out_hbm, sems, slot, j, h):
    return pltpu.make_async_copy(
        scr_ref.at[slot, pl.ds(h * TH, TH)],
        out_hbm.at[pl.ds(j * TV + h * TH, TH)],
        sems.at[slot, h],
    )


def _proj_body(emb_ref, w_ref, b_ref, out_hbm, embn_ref, scr_ref, sems):
    j = pl.program_id(0)
    nj = pl.num_programs(0)
    slot = jax.lax.rem(j, 2)

    @pl.when(j == 0)
    def _():
        e = emb_ref[...]
        ss = jnp.sum(e * e, axis=1, keepdims=True)
        norm = jnp.sqrt(ss)
        scale = jnp.where(norm > MAX_NORM, MAX_NORM / jnp.maximum(norm, 1e-12), 1.0)
        embn_ref[...] = e * scale

    x = lax.dot_general(
        w_ref[...], embn_ref[...], (((1,), (1,)), ((), ())),
        preferred_element_type=jnp.float32,
    )
    bias = b_ref[...]

    @pl.when(j >= 2)
    def _():
        for h in range(NSPLIT):
            _out_copy(scr_ref, out_hbm, sems, slot, j - 2, h).wait()

    scr_ref[slot] = x.reshape(TV, L, B) + bias[:, :, None]
    for h in range(NSPLIT):
        _out_copy(scr_ref, out_hbm, sems, slot, j, h).start()

    @pl.when(j == nj - 1)
    def _():
        for h in range(NSPLIT):
            _out_copy(scr_ref, out_hbm, sems, 1 - slot, j - 1, h).wait()
            _out_copy(scr_ref, out_hbm, sems, slot, j, h).wait()


def _projection(emb, W, b2):
    return pl.pallas_call(
        _proj_body,
        grid=(V // TV,),
        in_specs=[
            pl.BlockSpec((L * B, D), lambda j: (0, 0)),
            pl.BlockSpec((TV, D), lambda j: (j, 0)),
            pl.BlockSpec((TV, 1), lambda j: (j, 0)),
        ],
        out_specs=pl.BlockSpec(memory_space=pltpu.MemorySpace.HBM),
        out_shape=jax.ShapeDtypeStruct((V, L, B), jnp.float32),
        scratch_shapes=[
            pltpu.VMEM((L * B, D), jnp.float32),
            pltpu.VMEM((2, TV, L, B), jnp.float32),
            pltpu.SemaphoreType.DMA((2, NSPLIT)),
        ],
    )(emb, W, b2)


def kernel(inputs, dummy, table, W, b):
    idx_flat = inputs.T.reshape(-1).astype(jnp.int32)
    emb = _sc_gather(table, idx_flat)
    out_lvb = _projection(emb, W, b.reshape(V, 1))
    return (jnp.transpose(out_lvb, (2, 0, 1)), dummy)


# speedup vs baseline: 1.1423x; 1.1423x over previous
import jax
import jax.numpy as jnp
from jax import lax
from jax.experimental import pallas as pl
from jax.experimental.pallas import tpu as pltpu
from jax.experimental.pallas import tpu_sc as plsc

D = 128
L = 8
B = 128
V = 100000
MAX_NORM = 1.0

NC, NS = 2, 16
NW = NC * NS

TV = 4000


def _gather_body(table_hbm, idx_hbm, out_hbm, idx_v, rows_v, sem):
    wid = lax.axis_index("s") * NC + lax.axis_index("c")
    n = idx_v.shape[0]
    base = wid * n
    pltpu.sync_copy(idx_hbm.at[pl.ds(base, n)], idx_v)
    pltpu.async_copy(table_hbm.at[idx_v], rows_v, sem).wait()
    pltpu.sync_copy(rows_v, out_hbm.at[pl.ds(base, n)])


def _sc_gather(table, idx_flat):
    n_tok = idx_flat.shape[0]
    per_w = n_tok // NW
    mesh = plsc.VectorSubcoreMesh(
        core_axis_name="c", subcore_axis_name="s", num_cores=NC, num_subcores=NS
    )
    return pl.kernel(
        _gather_body,
        out_type=jax.ShapeDtypeStruct((n_tok, D), jnp.float32),
        mesh=mesh,
        scratch_types=[
            pltpu.VMEM((per_w,), jnp.int32),
            pltpu.VMEM((per_w, D), jnp.float32),
            pltpu.SemaphoreType.DMA,
        ],
    )(table, idx_flat)


def _proj_body(emb_ref, w_ref, b_ref, out_ref, embn_ref):
    j = pl.program_id(0)

    @pl.when(j == 0)
    def _():
        e = emb_ref[...]
        ss = jnp.sum(e * e, axis=1, keepdims=True)
        norm = jnp.sqrt(ss)
        scale = jnp.where(norm > MAX_NORM, MAX_NORM / jnp.maximum(norm, 1e-12), 1.0)
        embn_ref[...] = e * scale

    x = lax.dot_general(
        w_ref[...], embn_ref[...], (((1,), (1,)), ((), ())),
        preferred_element_type=jnp.float32,
    )
    bias = b_ref[...]
    out_ref[...] = x.reshape(TV, L, B) + bias[:, :, None]


def _projection(emb, W, b2):
    return pl.pallas_call(
        _proj_body,
        grid=(V // TV,),
        in_specs=[
            pl.BlockSpec((L * B, D), lambda j: (0, 0)),
            pl.BlockSpec((TV, D), lambda j: (j, 0)),
            pl.BlockSpec((TV, 1), lambda j: (j, 0)),
        ],
        out_specs=pl.BlockSpec((TV, L, B), lambda j: (j, 0, 0)),
        out_shape=jax.ShapeDtypeStruct((V, L, B), jnp.float32),
        scratch_shapes=[pltpu.VMEM((L * B, D), jnp.float32)],
    )(emb, W, b2)


def kernel(inputs, dummy, table, W, b):
    idx_flat = inputs.T.reshape(-1).astype(jnp.int32)
    emb = _sc_gather(table, idx_flat)
    out_lvb = _projection(emb, W, b.reshape(V, 1))
    return (jnp.transpose(out_lvb, (2, 0, 1)), dummy)
